# baseline (speedup 1.0000x reference)
import functools
import jax
import jax.numpy as jnp
from jax import lax
from jax.experimental import pallas as pl
from jax.experimental.pallas import tpu as pltpu, tpu_sc as plsc

N_PTS = 200000
N_BINS = 10000
CAP = 32
NW = 16
VECS_FULL = 832
VECS_LAST = 20
PTS_FULL = VECS_FULL * 16
PTS_LAST = VECS_LAST * 16
CHUNK_PTS = 1024
NCH_FULL = PTS_FULL // CHUNK_PTS
SUB_FULL = 8
SUB_LAST = 3
BIN_OWN = 640
BIN_OWN_LAST = 400
BIN_BLK = 640
BIN_HALF = 320
BIN_PAD = 10240
ZROWS = 400
P4CH = 1000
CO_FULL = 1888
CO_LAST = 1680
VOXEL_XY = 0.01

_mesh = plsc.VectorSubcoreMesh(core_axis_name="c", subcore_axis_name="s",
                               num_cores=1)


def _body(pts_hbm, zf_hbm, zi_hbm, zh_hbm, out_hbm, cc_hbm,
          vp_hbm, vco_hbm, vct_hbm,
          hist_sp, base_sp, nid_sp, np_sp,
          ptc_v, lin_v, rank_v, hb_v, hblk_v, tot_v, cc_v, zf_v,
          idx_v, stage_v, p4i_v, p4o_v, ccr_v):
    wid = lax.axis_index("s")
    iota = lax.iota(jnp.int32, 16)
    is_last = wid == NW - 1
    nvec = jnp.where(is_last, VECS_LAST, VECS_FULL)
    base_pt = wid * PTS_FULL
    own0 = wid * BIN_OWN

    pltpu.sync_copy(zf_hbm, zf_v)

    def z_body(k, zc):
        pltpu.sync_copy(zf_v, out_hbm.at[pl.ds(wid * 20000 + k * ZROWS,
                                               ZROWS)])
        return zc

    lax.fori_loop(0, 50, z_body, 0)
    own_w = jnp.where(is_last, BIN_OWN_LAST, BIN_OWN)
    pltpu.sync_copy(zi_hbm, cc_v)
    for k in range(5):
        pltpu.sync_copy(cc_v.at[pl.ds(0, 125)],
                        cc_hbm.at[pl.ds(wid * 625 + k * 125, 125)])
    pltpu.sync_copy(zh_hbm, hb_v)

    zero16 = jnp.zeros((16,), jnp.int32)
    one16 = jnp.ones((16,), jnp.int32)

    def load_chunk(c):
        @pl.when(jnp.logical_not(is_last))
        def _():
            pltpu.sync_copy(pts_hbm.at[pl.ds(base_pt + c * CHUNK_PTS,
                                             CHUNK_PTS)],
                            ptc_v.at[pl.ds(0, CHUNK_PTS)])

        @pl.when(is_last)
        def _():
            pltpu.sync_copy(pts_hbm.at[pl.ds(base_pt, PTS_LAST)],
                            ptc_v.at[pl.ds(0, PTS_LAST)])

    def p1_chunk(c, carry):
        load_chunk(c)

        def p1_body(i, carry2):
            row0 = i * 16
            x = plsc.load_gather(ptc_v, [row0 + iota, zero16])
            y = plsc.load_gather(ptc_v, [row0 + iota, one16])
            cx = (x / jnp.float32(VOXEL_XY)).astype(jnp.int32)
            cy = (y / jnp.float32(VOXEL_XY)).astype(jnp.int32)
            lin = jnp.clip(cx * 100 + cy, 0, N_BINS - 1)
            old = plsc.load_gather(hb_v, [lin])
            occ, lastm = plsc.scan_count(lin)
            lin_v[pl.ds(c * CHUNK_PTS + row0, 16)] = lin
            rank_v[pl.ds(c * CHUNK_PTS + row0, 16)] = old + occ - 1
            plsc.store_scatter(hb_v, [lin], old + occ, mask=lastm)
            return carry2

        nv_here = jnp.minimum(nvec - c * (CHUNK_PTS // 16), CHUNK_PTS // 16)
        lax.fori_loop(0, nv_here, p1_body, 0)
        return carry

    nch = jnp.where(is_last, 1, NCH_FULL)
    lax.fori_loop(0, nch, p1_chunk, 0)

    pltpu.sync_copy(hb_v, hist_sp.at[wid])
    plsc.subcore_barrier()

    for h in range(2):
        hoff = own0 + h * BIN_HALF
        for wp in range(NW):
            pltpu.sync_copy(hist_sp.at[wp, pl.ds(hoff, BIN_HALF)],
                            hblk_v.at[wp])
        def pfx_body(j, pc):
            acc = zero16
            for wp in range(NW):
                t = hblk_v[wp, pl.ds(j * 16, 16)]
                hblk_v[wp, pl.ds(j * 16, 16)] = acc
                acc = acc + t
            tot_v[pl.ds(h * BIN_HALF + j * 16, 16)] = acc
            return pc

        lax.fori_loop(0, BIN_HALF // 16, pfx_body, 0)
        for wp in range(NW):
            pltpu.sync_copy(hblk_v.at[wp],
                            base_sp.at[wp, pl.ds(hoff, BIN_HALF)])

    def npres_body(j, npres_c):
        l = j * 16 + iota
        tot16 = tot_v[pl.ds(j * 16, 16)]
        p = (tot16 > 0) & (l < own_w)
        return npres_c + p.astype(jnp.int32)

    npres = lax.fori_loop(0, BIN_BLK // 16, npres_body, zero16)
    npresent = jnp.sum(npres)
    idx_v[0, pl.ds(0, 16)] = jnp.full((16,), npresent, jnp.int32)
    pltpu.sync_copy(idx_v.at[0, pl.ds(0, 16)], np_sp.at[wid])
    plsc.subcore_barrier()

    for wp in range(NW):
        pltpu.sync_copy(np_sp.at[wp], hblk_v.at[wp, pl.ds(0, 16)])
    s_base = jnp.int32(0)
    for wp in range(NW):
        v = jnp.max(hblk_v[wp, pl.ds(0, 16)])
        s_base = s_base + jnp.where(jnp.int32(wp) < wid, v, jnp.int32(0))

    def cc_body(c5, carry):
        for j8 in range(8):
            j = c5 * 8 + j8
            l = j * 16 + iota
            binv = own0 + l
            tot16 = tot_v[pl.ds(j * 16, 16)]
            p = (tot16 > 0) & (l < own_w)
            pi = p.astype(jnp.int32)
            incl = plsc.cumsum(pi)
            nid16 = carry + incl - pi
            carry = carry + jnp.full((16,), jnp.max(incl), jnp.int32)
            hb_v[pl.ds(j * 16, 16)] = nid16
            rows = j8 * 16 + iota
            plsc.store_scatter(cc_v, [rows, zero16], binv // 100)
            plsc.store_scatter(cc_v, [rows, one16], binv % 100)
            plsc.store_scatter(cc_v, [rows, 3 * one16],
                               jnp.minimum(tot16, CAP))
            idx_v[0, pl.ds(j8 * 16, 16)] = jnp.where(p, nid16, -1)
        pltpu.sync_copy(
            cc_v.at[pl.ds(0, 128)],
            cc_hbm.at[plsc.Indices(idx_v.at[0], ignored_value=-1)])
        return carry

    lax.fori_loop(0, BIN_BLK // 128, cc_body,
                  jnp.full((16,), s_base, jnp.int32))
    pltpu.sync_copy(hb_v.at[pl.ds(0, BIN_BLK)],
                    nid_sp.at[pl.ds(own0, BIN_BLK)])
    plsc.subcore_barrier()

    pltpu.sync_copy(base_sp.at[wid], hb_v)

    def comb_body(b, bc):
        pltpu.sync_copy(nid_sp.at[pl.ds(b * BIN_BLK, BIN_BLK)],
                        tot_v.at[pl.ds(0, BIN_BLK)])

        def comb_j(j, jc):
            off = b * BIN_BLK + j * 16
            d = (tot_v[pl.ds(j * 16, 16)] << 18) | hb_v[pl.ds(off, 16)]
            hb_v[pl.ds(off, 16)] = d
            return jc

        lax.fori_loop(0, BIN_BLK // 16, comb_j, 0)
        return bc

    lax.fori_loop(0, BIN_PAD // BIN_BLK, comb_body, 0)
    pltpu.sync_copy(zf_hbm.at[pl.ds(0, 128)], stage_v)

    def p3_chunk(c, carry):
        load_chunk(c)

        def p3_sub(s, carry2):
            for v8 in range(8):
                lv = s * 8 + v8
                g = c * (CHUNK_PTS // 16) + lv
                row0 = lv * 16
                grow0 = c * CHUNK_PTS + row0
                lin = jnp.clip(lin_v[pl.ds(grow0, 16)], 0, N_BINS - 1)
                d = plsc.load_gather(hb_v, [lin])
                r = rank_v[pl.ds(grow0, 16)] + (d & 0x3FFFF)
                nid = lax.shift_right_logical(d, 18)
                slot = nid * CAP + r
                ok = (r < CAP) & (g < nvec)
                idx_v[0, pl.ds(v8 * 16, 16)] = jnp.where(ok, slot, -1)
                rows = v8 * 16 + iota
                for col in range(4):
                    colv = jnp.full((16,), col, jnp.int32)
                    f = plsc.load_gather(ptc_v, [row0 + iota, colv])
                    plsc.store_scatter(stage_v, [rows, colv], f)
            pltpu.sync_copy(
                stage_v,
                out_hbm.at[plsc.Indices(idx_v.at[0], ignored_value=-1)])
            return carry2

        nsub = jnp.where(is_last, SUB_LAST, SUB_FULL)
        lax.fori_loop(0, nsub, p3_sub, 0)
        return carry

    nch3 = jnp.where(is_last, 1, NCH_FULL)
    lax.fori_loop(0, nch3, p3_chunk, 0)
    plsc.subcore_barrier()

    def p4_chunk(k, kc):
        r0 = wid * 20000 + k * P4CH
        pltpu.sync_copy(out_hbm.at[pl.ds(r0, P4CH)], p4i_v)

        def p4_j(j, jc):
            t = j * 16 + iota
            f = plsc.load_gather(p4i_v, [t >> 2, t & 3])
            p4o_v[pl.ds(j * 16, 16)] = f
            return jc

        lax.fori_loop(0, P4CH * 4 // 16, p4_j, 0)
        pltpu.sync_copy(p4o_v, vp_hbm.at[pl.ds(r0 * 4, P4CH * 4)])
        return kc

    lax.fori_loop(0, 20000 // P4CH, p4_chunk, 0)

    pltpu.sync_copy(cc_hbm.at[pl.ds(own0, BIN_BLK)], ccr_v)

    def ct_j(j, jc):
        t = j * 16 + iota
        tot_v[pl.ds(j * 16, 16)] = plsc.load_gather(ccr_v, [t, 3 * one16])
        return jc

    lax.fori_loop(0, BIN_BLK // 16, ct_j, 0)

    @pl.when(jnp.logical_not(is_last))
    def _():
        pltpu.sync_copy(tot_v.at[pl.ds(0, BIN_BLK)],
                        vct_hbm.at[pl.ds(own0, BIN_BLK)])

    @pl.when(is_last)
    def _():
        pltpu.sync_copy(tot_v.at[pl.ds(0, BIN_OWN_LAST)],
                        vct_hbm.at[pl.ds(own0, BIN_OWN_LAST)])

    e0 = wid * CO_FULL
    a0 = e0 // 3
    pltpu.sync_copy(cc_hbm.at[pl.ds(a0, BIN_BLK)], ccr_v)

    def co_j(j, jc):
        e = e0 + j * 16 + iota
        b = e // 3 - a0
        lin_v[pl.ds(j * 16, 16)] = plsc.load_gather(ccr_v, [b, e % 3])
        return jc

    nco = jnp.where(is_last, CO_LAST // 16, CO_FULL // 16)
    lax.fori_loop(0, nco, co_j, 0)

    @pl.when(jnp.logical_not(is_last))
    def _():
        pltpu.sync_copy(lin_v.at[pl.ds(0, CO_FULL)],
                        vco_hbm.at[pl.ds(e0, CO_FULL)])

    @pl.when(is_last)
    def _():
        pltpu.sync_copy(lin_v.at[pl.ds(0, CO_LAST)],
                        vco_hbm.at[pl.ds(e0, CO_LAST)])


@functools.partial(
    pl.kernel,
    out_type=[
        jax.ShapeDtypeStruct((320000, 16), jnp.float32),
        jax.ShapeDtypeStruct((BIN_PAD, 16), jnp.int32),
        jax.ShapeDtypeStruct((320000 * 4,), jnp.float32),
        jax.ShapeDtypeStruct((30000,), jnp.int32),
        jax.ShapeDtypeStruct((N_BINS,), jnp.int32),
    ],
    mesh=_mesh,
    compiler_params=pltpu.CompilerParams(needs_layout_passes=False,
                                         use_tc_tiling_on_sc=False,
                                         skip_device_barrier=True),
    scratch_types=[
        pltpu.VMEM_SHARED((NW, BIN_PAD), jnp.int32),
        pltpu.VMEM_SHARED((NW, BIN_PAD), jnp.int32),
        pltpu.VMEM_SHARED((BIN_PAD,), jnp.int32),
        pltpu.VMEM_SHARED((NW, 16), jnp.int32),
        pltpu.VMEM((CHUNK_PTS, 4), jnp.float32),
        pltpu.VMEM((PTS_FULL,), jnp.int32),
        pltpu.VMEM((PTS_FULL,), jnp.int32),
        pltpu.VMEM((BIN_PAD,), jnp.int32),
        pltpu.VMEM((NW, BIN_HALF), jnp.int32),
        pltpu.VMEM((BIN_BLK,), jnp.int32),
        pltpu.VMEM((128, 16), jnp.int32),
        pltpu.VMEM((ZROWS, 16), jnp.float32),
        pltpu.VMEM((1, 128), jnp.int32),
        pltpu.VMEM((128, 16), jnp.float32),
        pltpu.VMEM((P4CH, 16), jnp.float32),
        pltpu.VMEM((P4CH * 4,), jnp.float32),
        pltpu.VMEM((BIN_BLK, 16), jnp.int32),
    ],
)
def _vox_call(pts_hbm, zf_hbm, zi_hbm, zh_hbm, out_hbm, cc_hbm,
              vp_hbm, vco_hbm, vct_hbm, *refs):
    _body(pts_hbm, zf_hbm, zi_hbm, zh_hbm, out_hbm, cc_hbm,
          vp_hbm, vco_hbm, vct_hbm, *refs)


def kernel(input):
    zf = jnp.zeros((ZROWS, 16), jnp.float32)
    zi = jnp.zeros((128, 16), jnp.int32)
    zh = jnp.zeros((BIN_PAD,), jnp.int32)
    _, _, vp, vco, vct = _vox_call(input, zf, zi, zh)
    return (vp.reshape(N_BINS, CAP, 4), vco.reshape(N_BINS, 3), vct)


# speedup vs baseline: 1.2612x; 1.2612x over previous
import functools
import jax
import jax.numpy as jnp
from jax import lax
from jax.experimental import pallas as pl
from jax.experimental.pallas import tpu as pltpu, tpu_sc as plsc

N_PTS = 200000
N_BINS = 10000
CAP = 32
NW = 16
VECS_FULL = 832
VECS_LAST = 20
PTS_FULL = VECS_FULL * 16
PTS_LAST = VECS_LAST * 16
CHUNK_PTS = 1024
NCH_FULL = PTS_FULL // CHUNK_PTS
SUB_FULL = 8
SUB_LAST = 3
BIN_OWN = 640
BIN_OWN_LAST = 400
BIN_BLK = 640
BIN_HALF = 320
BIN_PAD = 10240
ZROWS = 400
VOXEL_XY = 0.01

_mesh = plsc.VectorSubcoreMesh(core_axis_name="c", subcore_axis_name="s",
                               num_cores=1)


def _body(pts_hbm, zf_hbm, zi_hbm, zh_hbm, out_hbm, cc_hbm,
          hist_sp, base_sp, nid_sp, np_sp,
          ptc_v, lin_v, rank_v, hb_v, hblk_v, tot_v, cc_v, zf_v,
          idx_v, stage_v, zsem):
    wid = lax.axis_index("s")
    iota = lax.iota(jnp.int32, 16)
    is_last = wid == NW - 1
    nvec = jnp.where(is_last, VECS_LAST, VECS_FULL)
    base_pt = wid * PTS_FULL
    own0 = wid * BIN_OWN

    pltpu.sync_copy(zf_hbm, zf_v)

    zd = []
    for k in range(50):
        zd.append(pltpu.async_copy(
            zf_v, out_hbm.at[pl.ds(wid * 20000 + k * ZROWS, ZROWS)], zsem))
    for d in zd:
        d.wait()
    own_w = jnp.where(is_last, BIN_OWN_LAST, BIN_OWN)
    pltpu.sync_copy(zi_hbm, cc_v)
    for k in range(5):
        pltpu.sync_copy(cc_v.at[pl.ds(0, 125)],
                        cc_hbm.at[pl.ds(wid * 625 + k * 125, 125)])
    pltpu.sync_copy(zh_hbm, hb_v)

    zero16 = jnp.zeros((16,), jnp.int32)
    one16 = jnp.ones((16,), jnp.int32)

    def load_chunk(c):
        @pl.when(jnp.logical_not(is_last))
        def _():
            pltpu.sync_copy(pts_hbm.at[pl.ds(base_pt + c * CHUNK_PTS,
                                             CHUNK_PTS)],
                            ptc_v.at[pl.ds(0, CHUNK_PTS)])

        @pl.when(is_last)
        def _():
            pltpu.sync_copy(pts_hbm.at[pl.ds(base_pt, PTS_LAST)],
                            ptc_v.at[pl.ds(0, PTS_LAST)])

    def p1_chunk(c, carry):
        load_chunk(c)

        def p1_body(i, carry2):
            row0 = i * 16
            x = plsc.load_gather(ptc_v, [row0 + iota, zero16])
            y = plsc.load_gather(ptc_v, [row0 + iota, one16])
            cx = (x / jnp.float32(VOXEL_XY)).astype(jnp.int32)
            cy = (y / jnp.float32(VOXEL_XY)).astype(jnp.int32)
            lin = jnp.clip(cx * 100 + cy, 0, N_BINS - 1)
            old = plsc.load_gather(hb_v, [lin])
            occ, lastm = plsc.scan_count(lin)
            lin_v[pl.ds(c * CHUNK_PTS + row0, 16)] = lin
            rank_v[pl.ds(c * CHUNK_PTS + row0, 16)] = old + occ - 1
            plsc.store_scatter(hb_v, [lin], old + occ, mask=lastm)
            return carry2

        nv_here = jnp.minimum(nvec - c * (CHUNK_PTS // 16), CHUNK_PTS // 16)
        lax.fori_loop(0, nv_here, p1_body, 0)
        return carry

    nch = jnp.where(is_last, 1, NCH_FULL)
    lax.fori_loop(0, nch, p1_chunk, 0)

    pltpu.sync_copy(hb_v, hist_sp.at[wid])
    plsc.subcore_barrier()

    for h in range(2):
        hoff = own0 + h * BIN_HALF
        for wp in range(NW):
            pltpu.sync_copy(hist_sp.at[wp, pl.ds(hoff, BIN_HALF)],
                            hblk_v.at[wp])
        def pfx_body(j, pc):
            acc = zero16
            for wp in range(NW):
                t = hblk_v[wp, pl.ds(j * 16, 16)]
                hblk_v[wp, pl.ds(j * 16, 16)] = acc
                acc = acc + t
            tot_v[pl.ds(h * BIN_HALF + j * 16, 16)] = acc
            return pc

        lax.fori_loop(0, BIN_HALF // 16, pfx_body, 0)
        for wp in range(NW):
            pltpu.sync_copy(hblk_v.at[wp],
                            base_sp.at[wp, pl.ds(hoff, BIN_HALF)])

    def npres_body(j, npres_c):
        l = j * 16 + iota
        tot16 = tot_v[pl.ds(j * 16, 16)]
        p = (tot16 > 0) & (l < own_w)
        return npres_c + p.astype(jnp.int32)

    npres = lax.fori_loop(0, BIN_BLK // 16, npres_body, zero16)
    npresent = jnp.sum(npres)
    idx_v[0, pl.ds(0, 16)] = jnp.full((16,), npresent, jnp.int32)
    pltpu.sync_copy(idx_v.at[0, pl.ds(0, 16)], np_sp.at[wid])
    plsc.subcore_barrier()

    for wp in range(NW):
        pltpu.sync_copy(np_sp.at[wp], hblk_v.at[wp, pl.ds(0, 16)])
    s_base = jnp.int32(0)
    for wp in range(NW):
        v = jnp.max(hblk_v[wp, pl.ds(0, 16)])
        s_base = s_base + jnp.where(jnp.int32(wp) < wid, v, jnp.int32(0))

    def cc_body(c5, carry):
        for j8 in range(8):
            j = c5 * 8 + j8
            l = j * 16 + iota
            binv = own0 + l
            tot16 = tot_v[pl.ds(j * 16, 16)]
            p = (tot16 > 0) & (l < own_w)
            pi = p.astype(jnp.int32)
            incl = plsc.cumsum(pi)
            nid16 = carry + incl - pi
            carry = carry + jnp.full((16,), jnp.max(incl), jnp.int32)
            hb_v[pl.ds(j * 16, 16)] = nid16
            rows = j8 * 16 + iota
            plsc.store_scatter(cc_v, [rows, zero16], binv // 100)
            plsc.store_scatter(cc_v, [rows, one16], binv % 100)
            plsc.store_scatter(cc_v, [rows, 3 * one16],
                               jnp.minimum(tot16, CAP))
            idx_v[0, pl.ds(j8 * 16, 16)] = jnp.where(p, nid16, -1)
        pltpu.sync_copy(
            cc_v.at[pl.ds(0, 128)],
            cc_hbm.at[plsc.Indices(idx_v.at[0], ignored_value=-1)])
        return carry

    lax.fori_loop(0, BIN_BLK // 128, cc_body,
                  jnp.full((16,), s_base, jnp.int32))
    pltpu.sync_copy(hb_v.at[pl.ds(0, BIN_BLK)],
                    nid_sp.at[pl.ds(own0, BIN_BLK)])
    plsc.subcore_barrier()

    pltpu.sync_copy(base_sp.at[wid], hb_v)

    def comb_body(b, bc):
        pltpu.sync_copy(nid_sp.at[pl.ds(b * BIN_BLK, BIN_BLK)],
                        tot_v.at[pl.ds(0, BIN_BLK)])

        def comb_j(j, jc):
            off = b * BIN_BLK + j * 16
            d = (tot_v[pl.ds(j * 16, 16)] << 18) | hb_v[pl.ds(off, 16)]
            hb_v[pl.ds(off, 16)] = d
            return jc

        lax.fori_loop(0, BIN_BLK // 16, comb_j, 0)
        return bc

    lax.fori_loop(0, BIN_PAD // BIN_BLK, comb_body, 0)
    pltpu.sync_copy(zf_hbm.at[pl.ds(0, 128)], stage_v)

    def p3_chunk(c, carry):
        load_chunk(c)

        def p3_sub(s, carry2):
            for v8 in range(8):
                lv = s * 8 + v8
                g = c * (CHUNK_PTS // 16) + lv
                row0 = lv * 16
                grow0 = c * CHUNK_PTS + row0
                lin = jnp.clip(lin_v[pl.ds(grow0, 16)], 0, N_BINS - 1)
                d = plsc.load_gather(hb_v, [lin])
                r = rank_v[pl.ds(grow0, 16)] + (d & 0x3FFFF)
                nid = lax.shift_right_logical(d, 18)
                slot = nid * CAP + r
                ok = (r < CAP) & (g < nvec)
                idx_v[0, pl.ds(v8 * 16, 16)] = jnp.where(ok, slot, -1)
                rows = v8 * 16 + iota
                for col in range(4):
                    colv = jnp.full((16,), col, jnp.int32)
                    f = plsc.load_gather(ptc_v, [row0 + iota, colv])
                    plsc.store_scatter(stage_v, [rows, colv], f)
            pltpu.sync_copy(
                stage_v,
                out_hbm.at[plsc.Indices(idx_v.at[0], ignored_value=-1)])
            return carry2

        nsub = jnp.where(is_last, SUB_LAST, SUB_FULL)
        lax.fori_loop(0, nsub, p3_sub, 0)
        return carry

    nch3 = jnp.where(is_last, 1, NCH_FULL)
    lax.fori_loop(0, nch3, p3_chunk, 0)


@functools.partial(
    pl.kernel,
    out_type=[
        jax.ShapeDtypeStruct((320000, 8), jnp.float32),
        jax.ShapeDtypeStruct((N_BINS, 16), jnp.int32),
    ],
    mesh=_mesh,
    compiler_params=pltpu.CompilerParams(needs_layout_passes=False,
                                         use_tc_tiling_on_sc=False,
                                         skip_device_barrier=True),
    scratch_types=[
        pltpu.VMEM_SHARED((NW, BIN_PAD), jnp.int32),
        pltpu.VMEM_SHARED((NW, BIN_PAD), jnp.int32),
        pltpu.VMEM_SHARED((BIN_PAD,), jnp.int32),
        pltpu.VMEM_SHARED((NW, 16), jnp.int32),
        pltpu.VMEM((CHUNK_PTS, 4), jnp.float32),
        pltpu.VMEM((PTS_FULL,), jnp.int32),
        pltpu.VMEM((PTS_FULL,), jnp.int32),
        pltpu.VMEM((BIN_PAD,), jnp.int32),
        pltpu.VMEM((NW, BIN_HALF), jnp.int32),
        pltpu.VMEM((BIN_BLK,), jnp.int32),
        pltpu.VMEM((128, 16), jnp.int32),
        pltpu.VMEM((ZROWS, 8), jnp.float32),
        pltpu.VMEM((1, 128), jnp.int32),
        pltpu.VMEM((128, 8), jnp.float32),
        pltpu.SemaphoreType.DMA,
    ],
)
def _vox_call(pts_hbm, zf_hbm, zi_hbm, zh_hbm, out_hbm, cc_hbm, *refs):
    _body(pts_hbm, zf_hbm, zi_hbm, zh_hbm, out_hbm, cc_hbm, *refs)


def kernel(input):
    zf = jnp.zeros((ZROWS, 8), jnp.float32)
    zi = jnp.zeros((128, 16), jnp.int32)
    zh = jnp.zeros((BIN_PAD,), jnp.int32)
    out, cc = _vox_call(input, zf, zi, zh)
    voxels_points = out[:, :4].reshape(N_BINS, CAP, 4)
    voxels_coords = cc[:, :3]
    num_points_per_voxel = cc[:, 3]
    return (voxels_points, voxels_coords, num_points_per_voxel)


# speedup vs baseline: 1.2927x; 1.0249x over previous
import functools
import jax
import jax.numpy as jnp
from jax import lax
from jax.experimental import pallas as pl
from jax.experimental.pallas import tpu as pltpu, tpu_sc as plsc

N_PTS = 200000
N_BINS = 10000
CAP = 32
NW = 16
VECS_FULL = 832
VECS_LAST = 20
PTS_FULL = VECS_FULL * 16
PTS_LAST = VECS_LAST * 16
CHUNK_PTS = 1024
NCH_FULL = PTS_FULL // CHUNK_PTS
SUB_FULL = 8
SUB_LAST = 3
BIN_OWN = 640
BIN_OWN_LAST = 400
BIN_BLK = 640
BIN_HALF = 320
BIN_PAD = 10240
ZROWS = 400
VOXEL_XY = 0.01

_mesh = plsc.VectorSubcoreMesh(core_axis_name="c", subcore_axis_name="s",
                               num_cores=1)


def _body(pts_hbm, zf_hbm, zi_hbm, zh_hbm, out_hbm, cc_hbm,
          hist_sp, base_sp, nid_sp, np_sp,
          ptc_v, lin_v, rank_v, hb_v, hblk_v, tot_v, cc_v, zf_v,
          idx_v, stage_v, zsem, stage2_v, idx2_v, ssem2):
    wid = lax.axis_index("s")
    iota = lax.iota(jnp.int32, 16)
    is_last = wid == NW - 1
    nvec = jnp.where(is_last, VECS_LAST, VECS_FULL)
    base_pt = wid * PTS_FULL
    own0 = wid * BIN_OWN

    pltpu.sync_copy(zf_hbm, zf_v)

    zd = []
    for k in range(50):
        zd.append(pltpu.async_copy(
            zf_v, out_hbm.at[pl.ds(wid * 20000 + k * ZROWS, ZROWS)], zsem))
    for d in zd:
        d.wait()
    own_w = jnp.where(is_last, BIN_OWN_LAST, BIN_OWN)
    pltpu.sync_copy(zi_hbm, cc_v)
    for k in range(5):
        pltpu.sync_copy(cc_v.at[pl.ds(0, 125)],
                        cc_hbm.at[pl.ds(wid * 625 + k * 125, 125)])
    pltpu.sync_copy(zh_hbm, hb_v)

    zero16 = jnp.zeros((16,), jnp.int32)
    one16 = jnp.ones((16,), jnp.int32)

    def load_chunk(c):
        @pl.when(jnp.logical_not(is_last))
        def _():
            pltpu.sync_copy(pts_hbm.at[pl.ds(base_pt + c * CHUNK_PTS,
                                             CHUNK_PTS)],
                            ptc_v.at[pl.ds(0, CHUNK_PTS)])

        @pl.when(is_last)
        def _():
            pltpu.sync_copy(pts_hbm.at[pl.ds(base_pt, PTS_LAST)],
                            ptc_v.at[pl.ds(0, PTS_LAST)])

    def p1_chunk(c, carry):
        load_chunk(c)

        def p1_body(i, carry2):
            row0 = i * 16
            x = plsc.load_gather(ptc_v, [row0 + iota, zero16])
            y = plsc.load_gather(ptc_v, [row0 + iota, one16])
            cx = (x / jnp.float32(VOXEL_XY)).astype(jnp.int32)
            cy = (y / jnp.float32(VOXEL_XY)).astype(jnp.int32)
            lin = jnp.clip(cx * 100 + cy, 0, N_BINS - 1)
            old = plsc.load_gather(hb_v, [lin])
            occ, lastm = plsc.scan_count(lin)
            lin_v[pl.ds(c * CHUNK_PTS + row0, 16)] = lin
            rank_v[pl.ds(c * CHUNK_PTS + row0, 16)] = old + occ - 1
            plsc.store_scatter(hb_v, [lin], old + occ, mask=lastm)
            return carry2

        nv_here = jnp.minimum(nvec - c * (CHUNK_PTS // 16), CHUNK_PTS // 16)
        lax.fori_loop(0, nv_here, p1_body, 0)
        return carry

    nch = jnp.where(is_last, 1, NCH_FULL)
    lax.fori_loop(0, nch, p1_chunk, 0)

    pltpu.sync_copy(hb_v, hist_sp.at[wid])
    plsc.subcore_barrier()

    for h in range(2):
        hoff = own0 + h * BIN_HALF
        for wp in range(NW):
            pltpu.sync_copy(hist_sp.at[wp, pl.ds(hoff, BIN_HALF)],
                            hblk_v.at[wp])
        def pfx_body(j, pc):
            acc = zero16
            for wp in range(NW):
                t = hblk_v[wp, pl.ds(j * 16, 16)]
                hblk_v[wp, pl.ds(j * 16, 16)] = acc
                acc = acc + t
            tot_v[pl.ds(h * BIN_HALF + j * 16, 16)] = acc
            return pc

        lax.fori_loop(0, BIN_HALF // 16, pfx_body, 0)
        for wp in range(NW):
            pltpu.sync_copy(hblk_v.at[wp],
                            base_sp.at[wp, pl.ds(hoff, BIN_HALF)])

    def npres_body(j, npres_c):
        l = j * 16 + iota
        tot16 = tot_v[pl.ds(j * 16, 16)]
        p = (tot16 > 0) & (l < own_w)
        return npres_c + p.astype(jnp.int32)

    npres = lax.fori_loop(0, BIN_BLK // 16, npres_body, zero16)
    npresent = jnp.sum(npres)
    idx_v[0, pl.ds(0, 16)] = jnp.full((16,), npresent, jnp.int32)
    pltpu.sync_copy(idx_v.at[0, pl.ds(0, 16)], np_sp.at[wid])
    plsc.subcore_barrier()

    for wp in range(NW):
        pltpu.sync_copy(np_sp.at[wp], hblk_v.at[wp, pl.ds(0, 16)])
    s_base = jnp.int32(0)
    for wp in range(NW):
        v = jnp.max(hblk_v[wp, pl.ds(0, 16)])
        s_base = s_base + jnp.where(jnp.int32(wp) < wid, v, jnp.int32(0))

    def cc_body(c5, carry):
        for j8 in range(8):
            j = c5 * 8 + j8
            l = j * 16 + iota
            binv = own0 + l
            tot16 = tot_v[pl.ds(j * 16, 16)]
            p = (tot16 > 0) & (l < own_w)
            pi = p.astype(jnp.int32)
            incl = plsc.cumsum(pi)
            nid16 = carry + incl - pi
            carry = carry + jnp.full((16,), jnp.max(incl), jnp.int32)
            hb_v[pl.ds(j * 16, 16)] = nid16
            rows = j8 * 16 + iota
            plsc.store_scatter(cc_v, [rows, zero16], binv // 100)
            plsc.store_scatter(cc_v, [rows, one16], binv % 100)
            plsc.store_scatter(cc_v, [rows, 3 * one16],
                               jnp.minimum(tot16, CAP))
            idx_v[0, pl.ds(j8 * 16, 16)] = jnp.where(p, nid16, -1)
        pltpu.sync_copy(
            cc_v.at[pl.ds(0, 128)],
            cc_hbm.at[plsc.Indices(idx_v.at[0], ignored_value=-1)])
        return carry

    lax.fori_loop(0, BIN_BLK // 128, cc_body,
                  jnp.full((16,), s_base, jnp.int32))
    pltpu.sync_copy(hb_v.at[pl.ds(0, BIN_BLK)],
                    nid_sp.at[pl.ds(own0, BIN_BLK)])
    plsc.subcore_barrier()

    pltpu.sync_copy(base_sp.at[wid], hb_v)

    def comb_body(b, bc):
        pltpu.sync_copy(nid_sp.at[pl.ds(b * BIN_BLK, BIN_BLK)],
                        tot_v.at[pl.ds(0, BIN_BLK)])

        def comb_j(j, jc):
            off = b * BIN_BLK + j * 16
            d = (tot_v[pl.ds(j * 16, 16)] << 18) | hb_v[pl.ds(off, 16)]
            hb_v[pl.ds(off, 16)] = d
            return jc

        lax.fori_loop(0, BIN_BLK // 16, comb_j, 0)
        return bc

    lax.fori_loop(0, BIN_PAD // BIN_BLK, comb_body, 0)
    pltpu.sync_copy(zf_hbm.at[pl.ds(0, 128)], stage_v)

    def p3_chunk(c, carry):
        load_chunk(c)
        bufs = ((stage_v, idx_v, zsem), (stage2_v, idx2_v, ssem2))

        def build_and_fire(s, stage_b, idx_b, sem_b):
            for v8 in range(8):
                lv = s * 8 + v8
                g = c * (CHUNK_PTS // 16) + lv
                row0 = lv * 16
                grow0 = c * CHUNK_PTS + row0
                lin = jnp.clip(lin_v[pl.ds(grow0, 16)], 0, N_BINS - 1)
                d = plsc.load_gather(hb_v, [lin])
                r = rank_v[pl.ds(grow0, 16)] + (d & 0x3FFFF)
                nid = lax.shift_right_logical(d, 18)
                slot = nid * CAP + r
                ok = (r < CAP) & (g < nvec)
                idx_b[0, pl.ds(v8 * 16, 16)] = jnp.where(ok, slot, -1)
                rows = v8 * 16 + iota
                for col in range(4):
                    colv = jnp.full((16,), col, jnp.int32)
                    f = plsc.load_gather(ptc_v, [row0 + iota, colv])
                    plsc.store_scatter(stage_b, [rows, colv], f)
            return pltpu.async_copy(
                stage_b,
                out_hbm.at[plsc.Indices(idx_b.at[0], ignored_value=-1)],
                sem_b)

        prev = None
        for st in range(8):
            cur = build_and_fire(st, *bufs[st % 2])
            if prev is not None:
                prev.wait()
            prev = cur
        prev.wait()
        return carry

    nch3 = jnp.where(is_last, 1, NCH_FULL)
    lax.fori_loop(0, nch3, p3_chunk, 0)


@functools.partial(
    pl.kernel,
    out_type=[
        jax.ShapeDtypeStruct((320000, 8), jnp.float32),
        jax.ShapeDtypeStruct((N_BINS, 16), jnp.int32),
    ],
    mesh=_mesh,
    compiler_params=pltpu.CompilerParams(needs_layout_passes=False,
                                         use_tc_tiling_on_sc=False,
                                         skip_device_barrier=True),
    scratch_types=[
        pltpu.VMEM_SHARED((NW, BIN_PAD), jnp.int32),
        pltpu.VMEM_SHARED((NW, BIN_PAD), jnp.int32),
        pltpu.VMEM_SHARED((BIN_PAD,), jnp.int32),
        pltpu.VMEM_SHARED((NW, 16), jnp.int32),
        pltpu.VMEM((CHUNK_PTS, 4), jnp.float32),
        pltpu.VMEM((PTS_FULL,), jnp.int32),
        pltpu.VMEM((PTS_FULL,), jnp.int32),
        pltpu.VMEM((BIN_PAD,), jnp.int32),
        pltpu.VMEM((NW, BIN_HALF), jnp.int32),
        pltpu.VMEM((BIN_BLK,), jnp.int32),
        pltpu.VMEM((128, 16), jnp.int32),
        pltpu.VMEM((ZROWS, 8), jnp.float32),
        pltpu.VMEM((1, 128), jnp.int32),
        pltpu.VMEM((128, 8), jnp.float32),
        pltpu.SemaphoreType.DMA,
        pltpu.VMEM((128, 8), jnp.float32),
        pltpu.VMEM((1, 128), jnp.int32),
        pltpu.SemaphoreType.DMA,
    ],
)
def _vox_call(pts_hbm, zf_hbm, zi_hbm, zh_hbm, out_hbm, cc_hbm, *refs):
    _body(pts_hbm, zf_hbm, zi_hbm, zh_hbm, out_hbm, cc_hbm, *refs)


def kernel(input):
    zf = jnp.zeros((ZROWS, 8), jnp.float32)
    zi = jnp.zeros((128, 16), jnp.int32)
    zh = jnp.zeros((BIN_PAD,), jnp.int32)
    out, cc = _vox_call(input, zf, zi, zh)
    voxels_points = out[:, :4].reshape(N_BINS, CAP, 4)
    voxels_coords = cc[:, :3]
    num_points_per_voxel = cc[:, 3]
    return (voxels_points, voxels_coords, num_points_per_voxel)


# speedup vs baseline: 1.3107x; 1.0139x over previous
import functools
import jax
import jax.numpy as jnp
from jax import lax
from jax.experimental import pallas as pl
from jax.experimental.pallas import tpu as pltpu, tpu_sc as plsc

N_PTS = 200000
N_BINS = 10000
CAP = 32
NW = 16
VECS_FULL = 832
VECS_LAST = 20
PTS_FULL = VECS_FULL * 16
PTS_LAST = VECS_LAST * 16
CHUNK_PTS = 1024
NCH_FULL = PTS_FULL // CHUNK_PTS
SUB_FULL = 8
SUB_LAST = 3
BIN_OWN = 640
BIN_OWN_LAST = 400
BIN_BLK = 640
BIN_HALF = 320
BIN_PAD = 10240
ZROWS = 400
VOXEL_XY = 0.01

_mesh = plsc.VectorSubcoreMesh(core_axis_name="c", subcore_axis_name="s",
                               num_cores=1)


def _body(pts_hbm, zf_hbm, zi_hbm, zh_hbm, out_hbm, cc_hbm,
          hist_sp, base_sp, nid_sp, np_sp,
          ptc_v, lin_v, rank_v, hb_v, hblk_v, tot_v, cc_v, zf_v,
          idx_v, stage_v, zsem, stage2_v, idx2_v, ssem2):
    wid = lax.axis_index("s")
    iota = lax.iota(jnp.int32, 16)
    is_last = wid == NW - 1
    nvec = jnp.where(is_last, VECS_LAST, VECS_FULL)
    base_pt = wid * PTS_FULL
    own0 = wid * BIN_OWN

    pltpu.sync_copy(zf_hbm, zf_v)

    zd = []
    for k in range(50):
        zd.append(pltpu.async_copy(
            zf_v, out_hbm.at[pl.ds(wid * 20000 + k * ZROWS, ZROWS)], zsem))
    for d in zd:
        d.wait()
    own_w = jnp.where(is_last, BIN_OWN_LAST, BIN_OWN)
    pltpu.sync_copy(zi_hbm, cc_v)
    for k in range(5):
        pltpu.sync_copy(cc_v.at[pl.ds(0, 125)],
                        cc_hbm.at[pl.ds(wid * 625 + k * 125, 125)])
    pltpu.sync_copy(zh_hbm, hb_v)

    zero16 = jnp.zeros((16,), jnp.int32)
    one16 = jnp.ones((16,), jnp.int32)

    def load_chunk(c):
        @pl.when(jnp.logical_not(is_last))
        def _():
            pltpu.sync_copy(pts_hbm.at[pl.ds(base_pt + c * CHUNK_PTS,
                                             CHUNK_PTS)],
                            ptc_v.at[pl.ds(0, CHUNK_PTS)])

        @pl.when(is_last)
        def _():
            pltpu.sync_copy(pts_hbm.at[pl.ds(base_pt, PTS_LAST)],
                            ptc_v.at[pl.ds(0, PTS_LAST)])

    def p1_chunk(c, carry):
        load_chunk(c)

        def p1_body(i, carry2):
            row0 = i * 16
            x = plsc.load_gather(ptc_v, [row0 + iota, zero16])
            y = plsc.load_gather(ptc_v, [row0 + iota, one16])
            cx = (x / jnp.float32(VOXEL_XY)).astype(jnp.int32)
            cy = (y / jnp.float32(VOXEL_XY)).astype(jnp.int32)
            lin = jnp.clip(cx * 100 + cy, 0, N_BINS - 1)
            old = plsc.load_gather(hb_v, [lin])
            occ, lastm = plsc.scan_count(lin)
            lin_v[pl.ds(c * CHUNK_PTS + row0, 16)] = lin
            rank_v[pl.ds(c * CHUNK_PTS + row0, 16)] = old + occ - 1
            plsc.store_scatter(hb_v, [lin], old + occ, mask=lastm)
            return carry2

        nv_here = jnp.minimum(nvec - c * (CHUNK_PTS // 16), CHUNK_PTS // 16)
        lax.fori_loop(0, nv_here, p1_body, 0)
        return carry

    nch = jnp.where(is_last, 1, NCH_FULL)
    lax.fori_loop(0, nch, p1_chunk, 0)

    pltpu.sync_copy(hb_v, hist_sp.at[wid])
    plsc.subcore_barrier()

    for h in range(2):
        hoff = own0 + h * BIN_HALF
        hd = [pltpu.async_copy(hist_sp.at[wp, pl.ds(hoff, BIN_HALF)],
                               hblk_v.at[wp], zsem) for wp in range(NW)]
        for d_ in hd:
            d_.wait()
        def pfx_body(j, pc):
            acc = zero16
            for wp in range(NW):
                t = hblk_v[wp, pl.ds(j * 16, 16)]
                hblk_v[wp, pl.ds(j * 16, 16)] = acc
                acc = acc + t
            tot_v[pl.ds(h * BIN_HALF + j * 16, 16)] = acc
            return pc

        lax.fori_loop(0, BIN_HALF // 16, pfx_body, 0)
        bd = [pltpu.async_copy(hblk_v.at[wp],
                               base_sp.at[wp, pl.ds(hoff, BIN_HALF)], zsem)
              for wp in range(NW)]
        for d_ in bd:
            d_.wait()

    def npres_body(j, npres_c):
        l = j * 16 + iota
        tot16 = tot_v[pl.ds(j * 16, 16)]
        p = (tot16 > 0) & (l < own_w)
        return npres_c + p.astype(jnp.int32)

    npres = lax.fori_loop(0, BIN_BLK // 16, npres_body, zero16)
    npresent = jnp.sum(npres)
    idx_v[0, pl.ds(0, 16)] = jnp.full((16,), npresent, jnp.int32)
    pltpu.sync_copy(idx_v.at[0, pl.ds(0, 16)], np_sp.at[wid])
    plsc.subcore_barrier()

    nd = [pltpu.async_copy(np_sp.at[wp], hblk_v.at[wp, pl.ds(0, 16)], zsem)
          for wp in range(NW)]
    for d_ in nd:
        d_.wait()
    s_base = jnp.int32(0)
    for wp in range(NW):
        v = jnp.max(hblk_v[wp, pl.ds(0, 16)])
        s_base = s_base + jnp.where(jnp.int32(wp) < wid, v, jnp.int32(0))

    def cc_body(c5, carry):
        for j8 in range(8):
            j = c5 * 8 + j8
            l = j * 16 + iota
            binv = own0 + l
            tot16 = tot_v[pl.ds(j * 16, 16)]
            p = (tot16 > 0) & (l < own_w)
            pi = p.astype(jnp.int32)
            incl = plsc.cumsum(pi)
            nid16 = carry + incl - pi
            carry = carry + jnp.full((16,), jnp.max(incl), jnp.int32)
            hb_v[pl.ds(j * 16, 16)] = nid16
            rows = j8 * 16 + iota
            plsc.store_scatter(cc_v, [rows, zero16], binv // 100)
            plsc.store_scatter(cc_v, [rows, one16], binv % 100)
            plsc.store_scatter(cc_v, [rows, 3 * one16],
                               jnp.minimum(tot16, CAP))
            idx_v[0, pl.ds(j8 * 16, 16)] = jnp.where(p, nid16, -1)
        pltpu.sync_copy(
            cc_v.at[pl.ds(0, 128)],
            cc_hbm.at[plsc.Indices(idx_v.at[0], ignored_value=-1)])
        return carry

    lax.fori_loop(0, BIN_BLK // 128, cc_body,
                  jnp.full((16,), s_base, jnp.int32))
    pltpu.sync_copy(hb_v.at[pl.ds(0, BIN_BLK)],
                    nid_sp.at[pl.ds(own0, BIN_BLK)])
    plsc.subcore_barrier()

    pltpu.sync_copy(base_sp.at[wid], hb_v)

    def comb_body(b, bc):
        pltpu.sync_copy(nid_sp.at[pl.ds(b * BIN_BLK, BIN_BLK)],
                        tot_v.at[pl.ds(0, BIN_BLK)])

        def comb_j(j, jc):
            off = b * BIN_BLK + j * 16
            d = (tot_v[pl.ds(j * 16, 16)] << 18) | hb_v[pl.ds(off, 16)]
            hb_v[pl.ds(off, 16)] = d
            return jc

        lax.fori_loop(0, BIN_BLK // 16, comb_j, 0)
        return bc

    lax.fori_loop(0, BIN_PAD // BIN_BLK, comb_body, 0)
    pltpu.sync_copy(zf_hbm.at[pl.ds(0, 128)], stage_v)

    def p3_chunk(c, carry):
        load_chunk(c)
        bufs = ((stage_v, idx_v, zsem), (stage2_v, idx2_v, ssem2))

        def build_and_fire(s, stage_b, idx_b, sem_b):
            for v8 in range(8):
                lv = s * 8 + v8
                g = c * (CHUNK_PTS // 16) + lv
                row0 = lv * 16
                grow0 = c * CHUNK_PTS + row0
                lin = jnp.clip(lin_v[pl.ds(grow0, 16)], 0, N_BINS - 1)
                d = plsc.load_gather(hb_v, [lin])
                r = rank_v[pl.ds(grow0, 16)] + (d & 0x3FFFF)
                nid = lax.shift_right_logical(d, 18)
                slot = nid * CAP + r
                ok = (r < CAP) & (g < nvec)
                idx_b[0, pl.ds(v8 * 16, 16)] = jnp.where(ok, slot, -1)
                rows = v8 * 16 + iota
                for col in range(4):
                    colv = jnp.full((16,), col, jnp.int32)
                    f = plsc.load_gather(ptc_v, [row0 + iota, colv])
                    plsc.store_scatter(stage_b, [rows, colv], f)
            return pltpu.async_copy(
                stage_b,
                out_hbm.at[plsc.Indices(idx_b.at[0], ignored_value=-1)],
                sem_b)

        prev = None
        for st in range(8):
            cur = build_and_fire(st, *bufs[st % 2])
            if prev is not None:
                prev.wait()
            prev = cur
        prev.wait()
        return carry

    nch3 = jnp.where(is_last, 1, NCH_FULL)
    lax.fori_loop(0, nch3, p3_chunk, 0)


@functools.partial(
    pl.kernel,
    out_type=[
        jax.ShapeDtypeStruct((320000, 8), jnp.float32),
        jax.ShapeDtypeStruct((N_BINS, 16), jnp.int32),
    ],
    mesh=_mesh,
    compiler_params=pltpu.CompilerParams(needs_layout_passes=False,
                                         use_tc_tiling_on_sc=False,
                                         skip_device_barrier=True),
    scratch_types=[
        pltpu.VMEM_SHARED((NW, BIN_PAD), jnp.int32),
        pltpu.VMEM_SHARED((NW, BIN_PAD), jnp.int32),
        pltpu.VMEM_SHARED((BIN_PAD,), jnp.int32),
        pltpu.VMEM_SHARED((NW, 16), jnp.int32),
        pltpu.VMEM((CHUNK_PTS, 4), jnp.float32),
        pltpu.VMEM((PTS_FULL,), jnp.int32),
        pltpu.VMEM((PTS_FULL,), jnp.int32),
        pltpu.VMEM((BIN_PAD,), jnp.int32),
        pltpu.VMEM((NW, BIN_HALF), jnp.int32),
        pltpu.VMEM((BIN_BLK,), jnp.int32),
        pltpu.VMEM((128, 16), jnp.int32),
        pltpu.VMEM((ZROWS, 8), jnp.float32),
        pltpu.VMEM((1, 128), jnp.int32),
        pltpu.VMEM((128, 8), jnp.float32),
        pltpu.SemaphoreType.DMA,
        pltpu.VMEM((128, 8), jnp.float32),
        pltpu.VMEM((1, 128), jnp.int32),
        pltpu.SemaphoreType.DMA,
    ],
)
def _vox_call(pts_hbm, zf_hbm, zi_hbm, zh_hbm, out_hbm, cc_hbm, *refs):
    _body(pts_hbm, zf_hbm, zi_hbm, zh_hbm, out_hbm, cc_hbm, *refs)


def kernel(input):
    zf = jnp.zeros((ZROWS, 8), jnp.float32)
    zi = jnp.zeros((128, 16), jnp.int32)
    zh = jnp.zeros((BIN_PAD,), jnp.int32)
    out, cc = _vox_call(input, zf, zi, zh)
    voxels_points = out[:, :4].reshape(N_BINS, CAP, 4)
    voxels_coords = cc[:, :3]
    num_points_per_voxel = cc[:, 3]
    return (voxels_points, voxels_coords, num_points_per_voxel)
